# Initial kernel scaffold; baseline (speedup 1.0000x reference)
import jax
import jax.numpy as jnp
from jax import lax
from jax.experimental import pallas as pl
from jax.experimental.pallas import tpu as pltpu
from jax.experimental.pallas import tpu_sc as plsc

N = 10000
NP = 10240
E = 160000
EP = 163840
NC = 2
NS = 16
L = 16
B = 128
ER = EP // B
RPS = NP // NS
MROW = 2048
GR = NP // MROW

_MESH = plsc.VectorSubcoreMesh(core_axis_name="c", subcore_axis_name="s",
                               num_cores=NC, num_subcores=NS)


def _rsqrt16(x):
    i = plsc.bitcast(x, jnp.int32)
    y = plsc.bitcast(jnp.int32(0x5F3759DF) - (i >> 1), jnp.float32)
    for _ in range(3):
        y = y * (1.5 - 0.5 * x * y * y)
    return y


def _norm_body(src_h, dst_h, ew_h, norm_h, selfw_h,
               dst_v, ew_v, src_v, dis_f, tmp_v, dis_v, sw_v, nrm_v,
               deg_sh, dis_sh):
    c = lax.axis_index("c")
    s = lax.axis_index("s")
    rps = ER // NS
    nb = rps // NC
    rb = s * rps
    pltpu.sync_copy(dst_h.at[pl.ds(rb, rps)], dst_v)
    pltpu.sync_copy(ew_h.at[pl.ds(rb, rps)], ew_v)
    pltpu.sync_copy(src_h.at[pl.ds(rb + c * nb, nb)], src_v)

    @pl.loop(0, RPS // L)
    def _z(k):
        tmp_v[pl.ds(k * L, L)] = jnp.zeros((L,), jnp.float32)
    pltpu.sync_copy(tmp_v, deg_sh.at[pl.ds(s * RPS, RPS)])
    plsc.subcore_barrier()

    @pl.loop(0, rps)
    def _deg(b):
        pltpu.sync_copy(ew_v.at[b], deg_sh.at[dst_v.at[b]], add=True)
    plsc.subcore_barrier()

    pltpu.sync_copy(deg_sh.at[pl.ds(s * RPS, RPS)], tmp_v)

    @pl.loop(0, RPS // L)
    def _dis(k):
        sl = pl.ds(k * L, L)
        r = _rsqrt16(tmp_v[sl] + 1.0)
        dis_v[sl] = r
        sw_v[sl] = r * r
    pltpu.sync_copy(dis_v, dis_sh.at[pl.ds(s * RPS, RPS)])

    @pl.when(c == 0)
    def _sw():
        pltpu.sync_copy(sw_v, selfw_h.at[pl.ds(s * RPS, RPS)])
    plsc.subcore_barrier()

    pltpu.sync_copy(dis_sh, dis_f)

    @pl.loop(0, nb)
    def _nrm(r):
        for k in range(B // L):
            sl = pl.ds(k * L, L)
            s16 = src_v[r, sl]
            d16 = dst_v[c * nb + r, sl]
            e16 = ew_v[c * nb + r, sl]
            nrm_v[r, sl] = (plsc.load_gather(dis_f, [s16]) * e16 *
                            plsc.load_gather(dis_f, [d16]))
    pltpu.sync_copy(nrm_v, norm_h.at[pl.ds(rb + c * nb, nb)])


_norm_call = pl.kernel(
    _norm_body,
    out_type=(jax.ShapeDtypeStruct((ER, B), jnp.float32),
              jax.ShapeDtypeStruct((NP,), jnp.float32)),
    mesh=_MESH,
    scratch_types=[
        pltpu.VMEM((ER // NS, B), jnp.int32),
        pltpu.VMEM((ER // NS, B), jnp.float32),
        pltpu.VMEM((ER // NS // NC, B), jnp.int32),
        pltpu.VMEM((NP,), jnp.float32),
        pltpu.VMEM((RPS,), jnp.float32),
        pltpu.VMEM((RPS,), jnp.float32),
        pltpu.VMEM((RPS,), jnp.float32),
        pltpu.VMEM((ER // NS // NC, B), jnp.float32),
        pltpu.VMEM_SHARED((NP,), jnp.float32),
        pltpu.VMEM_SHARED((NP,), jnp.float32),
    ],
)


def _make_prop(nch, cw, edge_split):
    nbs = (ER // NC if edge_split else ER) // NS
    npart = NC if edge_split else nch

    def body(yf_h, src_h, dst_h, nrm_h, z_h,
             src_v, gidx_v, dst_v, nrm_v, rows_v, acc_sh):
        c = lax.axis_index("c")
        s = lax.axis_index("s")
        rb = (c * (ER // NC) + s * nbs) if edge_split else (s * nbs)
        pltpu.sync_copy(src_h.at[pl.ds(rb, nbs)], src_v)
        pltpu.sync_copy(dst_h.at[pl.ds(rb, nbs)], dst_v)
        pltpu.sync_copy(nrm_h.at[pl.ds(rb, nbs)], nrm_v)

        for i in range(1 if edge_split else nch // NC):
            if edge_split:
                j = jnp.int32(0)
                zslot = c
            else:
                j = i * NC + c
                zslot = j

            @pl.loop(0, nbs)
            def _gi(b):
                for k in range(B // L):
                    sl = pl.ds(k * L, L)
                    gidx_v[b, sl] = src_v[b, sl] + j * NP

            @pl.loop(0, B)
            def _zr(r):
                for k in range(cw // L):
                    rows_v[r, pl.ds(k * L, L)] = jnp.zeros((L,), jnp.float32)
            for t in range(RPS // B):
                pltpu.sync_copy(rows_v, acc_sh.at[pl.ds(s * RPS + t * B, B), :])
            plsc.subcore_barrier()

            @pl.loop(0, nbs)
            def _edge(b):
                pltpu.sync_copy(yf_h.at[gidx_v.at[b]], rows_v)

                @pl.loop(0, B)
                def _scale(r):
                    w = nrm_v[b, r]
                    for k in range(cw // L):
                        sl = pl.ds(k * L, L)
                        rows_v[r, sl] = rows_v[r, sl] * w
                pltpu.sync_copy(rows_v, acc_sh.at[dst_v.at[b]], add=True)
            plsc.subcore_barrier()
            pltpu.sync_copy(acc_sh.at[pl.ds(s * RPS, RPS), :],
                            z_h.at[zslot, pl.ds(s * RPS, RPS), :])

    return pl.kernel(
        body,
        out_type=jax.ShapeDtypeStruct((npart, NP, cw), jnp.float32),
        mesh=_MESH,
        scratch_types=[
            pltpu.VMEM((nbs, B), jnp.int32),
            pltpu.VMEM((nbs, B), jnp.int32),
            pltpu.VMEM((nbs, B), jnp.int32),
            pltpu.VMEM((nbs, B), jnp.float32),
            pltpu.VMEM((B, cw), jnp.float32),
            pltpu.VMEM_SHARED((NP, cw), jnp.float32),
        ],
    )


_prop_512 = _make_prop(4, 128, False)
_prop_64 = _make_prop(1, 64, True)
_prop_256 = _make_prop(2, 128, False)


def _tc1(x, W):
    def body(x_ref, w_ref, o_ref):
        o_ref[0] = jnp.dot(x_ref[...], w_ref[...],
                           preferred_element_type=jnp.float32)

    return pl.pallas_call(
        body,
        grid=(GR, 4),
        in_specs=[pl.BlockSpec((MROW, 256), lambda i, j: (i, 0)),
                  pl.BlockSpec((256, 128), lambda i, j: (0, j))],
        out_specs=pl.BlockSpec((1, MROW, 128), lambda i, j: (j, i, 0)),
        out_shape=jax.ShapeDtypeStruct((4, NP, 128), jnp.float32),
    )(x, W)


def _tc_layer(z, y, sw, bc, W, nch_in, cw_in, nch_out, cw_out, parts, want_h):
    P = z.shape[0]

    def body(z_ref, y_ref, sw_ref, b_ref, w_ref, yo_ref, *h_ref):
        k = pl.program_id(2)
        zsum = z_ref[0] + z_ref[1] if parts else z_ref[0]
        h = zsum + sw_ref[...] * y_ref[0] + b_ref[...]
        if want_h:
            h_ref[0][...] = h

        @pl.when(k == 0)
        def _():
            yo_ref[...] = jnp.zeros_like(yo_ref)
        yo_ref[...] += jnp.dot(h, w_ref[...], preferred_element_type=jnp.float32)

    in_specs = [
        pl.BlockSpec((P, MROW, cw_in), lambda i, j, k: (0, i, 0)) if parts
        else pl.BlockSpec((1, MROW, cw_in), lambda i, j, k: (k, i, 0)),
        pl.BlockSpec((1, MROW, cw_in), lambda i, j, k: (k, i, 0)),
        pl.BlockSpec((MROW, 1), lambda i, j, k: (i, 0)),
        pl.BlockSpec((1, cw_in), lambda i, j, k: (k, 0)),
        pl.BlockSpec((cw_in, cw_out), lambda i, j, k: (k, j)),
    ]
    out_shape = [jax.ShapeDtypeStruct((nch_out, NP, cw_out), jnp.float32)]
    out_specs = [pl.BlockSpec((1, MROW, cw_out), lambda i, j, k: (j, i, 0))]
    if want_h:
        out_shape.append(jax.ShapeDtypeStruct((NP, nch_in * cw_in), jnp.float32))
        out_specs.append(pl.BlockSpec((MROW, cw_in), lambda i, j, k: (i, k)))
    res = pl.pallas_call(
        body,
        grid=(GR, nch_out, nch_in),
        in_specs=in_specs,
        out_specs=out_specs,
        out_shape=out_shape,
    )(z, y, sw, bc, W)
    return res if want_h else res[0]


def _tc_final(z, y, sw, bc, nch, cw):
    def body(z_ref, y_ref, sw_ref, b_ref, h_ref):
        h_ref[...] = z_ref[0] + sw_ref[...] * y_ref[0] + b_ref[...]

    return pl.pallas_call(
        body,
        grid=(GR, nch),
        in_specs=[pl.BlockSpec((1, MROW, cw), lambda i, k: (k, i, 0)),
                  pl.BlockSpec((1, MROW, cw), lambda i, k: (k, i, 0)),
                  pl.BlockSpec((MROW, 1), lambda i, k: (i, 0)),
                  pl.BlockSpec((1, cw), lambda i, k: (k, 0))],
        out_specs=pl.BlockSpec((MROW, cw), lambda i, k: (i, k)),
        out_shape=jax.ShapeDtypeStruct((NP, nch * cw), jnp.float32),
    )(z, y, sw, bc)


def kernel(features, edge_index, edge_weight, W1, b1, W2, b2, W3, b3, W4, b4):
    src = edge_index[0].astype(jnp.int32)
    dst = edge_index[1].astype(jnp.int32)
    ew = edge_weight.astype(jnp.float32)
    pad = EP - E
    src2 = jnp.concatenate([src, jnp.zeros((pad,), jnp.int32)]).reshape(ER, B)
    dst2 = jnp.concatenate([dst, jnp.zeros((pad,), jnp.int32)]).reshape(ER, B)
    ew2 = jnp.concatenate([ew, jnp.zeros((pad,), jnp.float32)]).reshape(ER, B)
    xp = jnp.pad(features, ((0, NP - N), (0, 0)))

    norm2, selfw = _norm_call(src2, dst2, ew2)
    sw = selfw.reshape(NP, 1)

    y1 = _tc1(xp, W1)
    z1 = _prop_512(y1.reshape(4 * NP, 128), src2, dst2, norm2)
    y2 = _tc_layer(z1, y1, sw, b1.reshape(4, 128), W2,
                   4, 128, 1, 64, parts=False, want_h=False)
    z2 = _prop_64(y2.reshape(NP, 64), src2, dst2, norm2)
    y3, h2 = _tc_layer(z2, y2, sw, b2.reshape(1, 64), W3,
                       1, 64, 4, 128, parts=True, want_h=True)
    z3 = _prop_512(y3.reshape(4 * NP, 128), src2, dst2, norm2)
    y4 = _tc_layer(z3, y3, sw, b3.reshape(4, 128), W4,
                   4, 128, 2, 128, parts=False, want_h=False)
    z4 = _prop_256(y4.reshape(2 * NP, 128), src2, dst2, norm2)
    h4 = _tc_final(z4, y4, sw, b4.reshape(2, 128), 2, 128)
    return h2[:N], h4[:N]


# speedup vs baseline: 2.7606x; 2.7606x over previous
import jax
import jax.numpy as jnp
from jax import lax
from jax.experimental import pallas as pl
from jax.experimental.pallas import tpu as pltpu
from jax.experimental.pallas import tpu_sc as plsc

N = 10000
NP = 10240
E = 160000
EP = 163840
NC = 2
NS = 16
L = 16
B = 128
ER = EP // B
RPS = NP // NS
MROW = 2048
GR = NP // MROW

_MESH = plsc.VectorSubcoreMesh(core_axis_name="c", subcore_axis_name="s",
                               num_cores=NC, num_subcores=NS)


def _rsqrt16(x):
    i = lax.bitcast_convert_type(x, jnp.int32)
    y = lax.bitcast_convert_type(jnp.int32(0x5F3759DF) - (i >> 1), jnp.float32)
    for _ in range(3):
        y = y * (1.5 - 0.5 * x * y * y)
    return y


def _norm_body(src_h, dst_h, ew_h, norm_h, selfw_h,
               dst_v, ew_v, src_v, dis_f, tmp_v, dis_v, sw_v, nrm_v,
               deg_sh, dis_sh):
    c = lax.axis_index("c")
    s = lax.axis_index("s")
    rps = ER // NS
    nb = rps // NC
    rb = s * rps
    pltpu.sync_copy(dst_h.at[pl.ds(rb, rps)], dst_v)
    pltpu.sync_copy(ew_h.at[pl.ds(rb, rps)], ew_v)
    pltpu.sync_copy(src_h.at[pl.ds(rb + c * nb, nb)], src_v)

    @pl.loop(0, RPS // L)
    def _z(k):
        tmp_v[pl.ds(k * L, L)] = jnp.zeros((L,), jnp.float32)
    pltpu.sync_copy(tmp_v, deg_sh.at[pl.ds(s * RPS, RPS)])
    plsc.subcore_barrier()

    @pl.loop(0, rps)
    def _deg(b):
        pltpu.sync_copy(ew_v.at[b], deg_sh.at[dst_v.at[b]], add=True)
    plsc.subcore_barrier()

    pltpu.sync_copy(deg_sh.at[pl.ds(s * RPS, RPS)], tmp_v)

    @pl.loop(0, RPS // L)
    def _dis(k):
        sl = pl.ds(k * L, L)
        r = _rsqrt16(tmp_v[sl] + 1.0)
        dis_v[sl] = r
        sw_v[sl] = r * r
    pltpu.sync_copy(dis_v, dis_sh.at[pl.ds(s * RPS, RPS)])

    @pl.when(c == 0)
    def _sw():
        pltpu.sync_copy(sw_v, selfw_h.at[pl.ds(s * RPS, RPS)])
    plsc.subcore_barrier()

    pltpu.sync_copy(dis_sh, dis_f)

    @pl.loop(0, nb)
    def _nrm(r):
        for k in range(B // L):
            sl = pl.ds(k * L, L)
            s16 = src_v[r, sl]
            d16 = dst_v[c * nb + r, sl]
            e16 = ew_v[c * nb + r, sl]
            nrm_v[r, sl] = (plsc.load_gather(dis_f, [s16]) * e16 *
                            plsc.load_gather(dis_f, [d16]))
    pltpu.sync_copy(nrm_v, norm_h.at[pl.ds(rb + c * nb, nb)])


_norm_call = pl.kernel(
    _norm_body,
    out_type=(jax.ShapeDtypeStruct((ER, B), jnp.float32),
              jax.ShapeDtypeStruct((NP,), jnp.float32)),
    mesh=_MESH,
    compiler_params=pltpu.CompilerParams(needs_layout_passes=False, use_tc_tiling_on_sc=False),
    scratch_types=[
        pltpu.VMEM((ER // NS, B), jnp.int32),
        pltpu.VMEM((ER // NS, B), jnp.float32),
        pltpu.VMEM((ER // NS // NC, B), jnp.int32),
        pltpu.VMEM((NP,), jnp.float32),
        pltpu.VMEM((RPS,), jnp.float32),
        pltpu.VMEM((RPS,), jnp.float32),
        pltpu.VMEM((RPS,), jnp.float32),
        pltpu.VMEM((ER // NS // NC, B), jnp.float32),
        pltpu.VMEM_SHARED((NP,), jnp.float32),
        pltpu.VMEM_SHARED((NP,), jnp.float32),
    ],
)


def _make_prop(nch, cw, edge_split):
    nbs = (ER // NC if edge_split else ER) // NS
    npart = NC if edge_split else nch

    def body(yf_h, src_h, dst_h, nrm_h, z_h,
             src_v, gidx_v, dst_v, nrm_v, rows_v, acc_sh):
        c = lax.axis_index("c")
        s = lax.axis_index("s")
        rb = (c * (ER // NC) + s * nbs) if edge_split else (s * nbs)
        pltpu.sync_copy(src_h.at[pl.ds(rb, nbs)], src_v)
        pltpu.sync_copy(dst_h.at[pl.ds(rb, nbs)], dst_v)
        pltpu.sync_copy(nrm_h.at[pl.ds(rb, nbs)], nrm_v)

        for i in range(1 if edge_split else nch // NC):
            if edge_split:
                j = jnp.int32(0)
                zslot = c
            else:
                j = i * NC + c
                zslot = j

            @pl.loop(0, nbs)
            def _gi(b):
                for k in range(B // L):
                    sl = pl.ds(k * L, L)
                    gidx_v[b, sl] = src_v[b, sl] + j * NP

            @pl.loop(0, B)
            def _zr(r):
                for k in range(cw // L):
                    rows_v[r, pl.ds(k * L, L)] = jnp.zeros((L,), jnp.float32)
            for t in range(RPS // B):
                pltpu.sync_copy(rows_v, acc_sh.at[pl.ds(s * RPS + t * B, B), :])
            plsc.subcore_barrier()

            @pl.loop(0, nbs)
            def _edge(b):
                pltpu.sync_copy(yf_h.at[gidx_v.at[b]], rows_v)

                @pl.loop(0, B // L)
                def _scale(g):
                    w16 = nrm_v[b, pl.ds(g * L, L)]
                    for t in range(L):
                        w = w16[t]
                        r = g * L + t
                        for k in range(cw // L):
                            sl = pl.ds(k * L, L)
                            rows_v[r, sl] = rows_v[r, sl] * w
                pltpu.sync_copy(rows_v, acc_sh.at[dst_v.at[b]], add=True)
            plsc.subcore_barrier()
            pltpu.sync_copy(acc_sh.at[pl.ds(s * RPS, RPS), :],
                            z_h.at[zslot, pl.ds(s * RPS, RPS), :])

    return pl.kernel(
        body,
        out_type=jax.ShapeDtypeStruct((npart, NP, cw), jnp.float32),
        mesh=_MESH,
        compiler_params=pltpu.CompilerParams(needs_layout_passes=False, use_tc_tiling_on_sc=False),
        scratch_types=[
            pltpu.VMEM((nbs, B), jnp.int32),
            pltpu.VMEM((nbs, B), jnp.int32),
            pltpu.VMEM((nbs, B), jnp.int32),
            pltpu.VMEM((nbs, B), jnp.float32),
            pltpu.VMEM((B, cw), jnp.float32),
            pltpu.VMEM_SHARED((NP, cw), jnp.float32),
        ],
    )


_prop_512 = _make_prop(8, 64, False)
_prop_64 = _make_prop(1, 64, True)
_prop_256 = _make_prop(4, 64, False)


def _tc1(x, W):
    nch = W.shape[1] // 64

    def body(x_ref, w_ref, o_ref):
        y = jnp.dot(x_ref[...], w_ref[...], preferred_element_type=jnp.float32)
        for jj in range(nch):
            o_ref[jj] = y[:, jj * 64:(jj + 1) * 64]

    return pl.pallas_call(
        body,
        grid=(GR,),
        in_specs=[pl.BlockSpec((MROW, 256), lambda i: (i, 0)),
                  pl.BlockSpec((256, W.shape[1]), lambda i: (0, 0))],
        out_specs=pl.BlockSpec((nch, MROW, 64), lambda i: (0, i, 0)),
        out_shape=jax.ShapeDtypeStruct((nch, NP, 64), jnp.float32),
    )(x, W)


def _tc_layer(z, y, sw, bc, W, nch_in, nch_out, parts, want_h):
    P = z.shape[0]
    din = nch_in * 64
    dout = nch_out * 64

    def body(z_ref, y_ref, sw_ref, b_ref, w_ref, yo_ref, *h_ref):
        hs = []
        for kk in range(nch_in):
            if parts:
                zsum = z_ref[0] + z_ref[1]
            else:
                zsum = z_ref[kk]
            hs.append(zsum + sw_ref[...] * y_ref[kk] + b_ref[kk])
        h = hs[0] if nch_in == 1 else jnp.concatenate(hs, axis=-1)
        if want_h:
            h_ref[0][...] = h
        yn = jnp.dot(h, w_ref[...], preferred_element_type=jnp.float32)
        for jj in range(nch_out):
            yo_ref[jj] = yn[:, jj * 64:(jj + 1) * 64]

    in_specs = [
        pl.BlockSpec((P, MROW, 64), lambda i: (0, i, 0)),
        pl.BlockSpec((nch_in, MROW, 64), lambda i: (0, i, 0)),
        pl.BlockSpec((MROW, 1), lambda i: (i, 0)),
        pl.BlockSpec((nch_in, 1, 64), lambda i: (0, 0, 0)),
        pl.BlockSpec((din, dout), lambda i: (0, 0)),
    ]
    out_shape = [jax.ShapeDtypeStruct((nch_out, NP, 64), jnp.float32)]
    out_specs = [pl.BlockSpec((nch_out, MROW, 64), lambda i: (0, i, 0))]
    if want_h:
        out_shape.append(jax.ShapeDtypeStruct((NP, din), jnp.float32))
        out_specs.append(pl.BlockSpec((MROW, din), lambda i: (i, 0)))
    res = pl.pallas_call(
        body,
        grid=(GR,),
        in_specs=in_specs,
        out_specs=out_specs,
        out_shape=out_shape,
    )(z, y, sw, bc, W)
    return res if want_h else res[0]


def _tc_final(z, y, sw, bc, nch):
    def body(z_ref, y_ref, sw_ref, b_ref, h_ref):
        hs = [z_ref[kk] + sw_ref[...] * y_ref[kk] + b_ref[kk]
              for kk in range(nch)]
        h_ref[...] = jnp.concatenate(hs, axis=-1)

    return pl.pallas_call(
        body,
        grid=(GR,),
        in_specs=[pl.BlockSpec((nch, MROW, 64), lambda i: (0, i, 0)),
                  pl.BlockSpec((nch, MROW, 64), lambda i: (0, i, 0)),
                  pl.BlockSpec((MROW, 1), lambda i: (i, 0)),
                  pl.BlockSpec((nch, 1, 64), lambda i: (0, 0, 0))],
        out_specs=pl.BlockSpec((MROW, nch * 64), lambda i: (i, 0)),
        out_shape=jax.ShapeDtypeStruct((NP, nch * 64), jnp.float32),
    )(z, y, sw, bc)


def kernel(features, edge_index, edge_weight, W1, b1, W2, b2, W3, b3, W4, b4):
    src = edge_index[0].astype(jnp.int32)
    dst = edge_index[1].astype(jnp.int32)
    ew = edge_weight.astype(jnp.float32)
    pad = EP - E
    src2 = jnp.concatenate([src, jnp.zeros((pad,), jnp.int32)]).reshape(ER, B)
    dst2 = jnp.concatenate([dst, jnp.zeros((pad,), jnp.int32)]).reshape(ER, B)
    ew2 = jnp.concatenate([ew, jnp.zeros((pad,), jnp.float32)]).reshape(ER, B)
    xp = jnp.pad(features, ((0, NP - N), (0, 0)))

    norm2, selfw = _norm_call(src2, dst2, ew2)
    sw = selfw.reshape(NP, 1)

    y1 = _tc1(xp, W1)
    z1 = _prop_512(y1.reshape(8 * NP, 64), src2, dst2, norm2)
    y2 = _tc_layer(z1, y1, sw, b1.reshape(8, 1, 64), W2,
                   8, 1, parts=False, want_h=False)
    z2 = _prop_64(y2.reshape(NP, 64), src2, dst2, norm2)
    y3, h2 = _tc_layer(z2, y2, sw, b2.reshape(1, 1, 64), W3,
                       1, 8, parts=True, want_h=True)
    z3 = _prop_512(y3.reshape(8 * NP, 64), src2, dst2, norm2)
    y4 = _tc_layer(z3, y3, sw, b3.reshape(8, 1, 64), W4,
                   8, 4, parts=False, want_h=False)
    z4 = _prop_256(y4.reshape(4 * NP, 64), src2, dst2, norm2)
    h4 = _tc_final(z4, y4, sw, b4.reshape(4, 1, 64), 4)
    return h2[:N], h4[:N]


# speedup vs baseline: 4.4934x; 1.6277x over previous
import jax
import jax.numpy as jnp
from jax import lax
from jax.experimental import pallas as pl
from jax.experimental.pallas import tpu as pltpu
from jax.experimental.pallas import tpu_sc as plsc

N = 10000
NP = 10240
E = 160000
EP = 163840
NC = 2
NS = 16
L = 16
B = 128
ER = EP // B
RPS = NP // NS
MROW = 2048
GR = NP // MROW

_MESH = plsc.VectorSubcoreMesh(core_axis_name="c", subcore_axis_name="s",
                               num_cores=NC, num_subcores=NS)


def _rsqrt16(x):
    i = lax.bitcast_convert_type(x, jnp.int32)
    y = lax.bitcast_convert_type(jnp.int32(0x5F3759DF) - (i >> 1), jnp.float32)
    for _ in range(3):
        y = y * (1.5 - 0.5 * x * y * y)
    return y


def _norm_body(src_h, dst_h, ew_h, norm_h, selfw_h,
               dst_v, ew_v, src_v, dis_f, tmp_v, dis_v, sw_v, nrm_v,
               deg_sh, dis_sh, dsem):
    c = lax.axis_index("c")
    s = lax.axis_index("s")
    rps = ER // NS
    nb = rps // NC
    rb = s * rps
    pltpu.sync_copy(dst_h.at[pl.ds(rb, rps)], dst_v)
    pltpu.sync_copy(ew_h.at[pl.ds(rb, rps)], ew_v)
    pltpu.sync_copy(src_h.at[pl.ds(rb + c * nb, nb)], src_v)

    @pl.loop(0, RPS // L)
    def _z(k):
        tmp_v[pl.ds(k * L, L)] = jnp.zeros((L,), jnp.float32)
    pltpu.sync_copy(tmp_v, deg_sh.at[pl.ds(s * RPS, RPS)])
    plsc.subcore_barrier()

    dd = [pltpu.async_copy(ew_v.at[b], deg_sh.at[dst_v.at[b]], dsem, add=True)
          for b in range(rps)]
    for d in dd:
        d.wait()
    plsc.subcore_barrier()

    pltpu.sync_copy(deg_sh.at[pl.ds(s * RPS, RPS)], tmp_v)

    @pl.loop(0, RPS // L)
    def _dis(k):
        sl = pl.ds(k * L, L)
        r = _rsqrt16(tmp_v[sl] + 1.0)
        dis_v[sl] = r
        sw_v[sl] = r * r
    pltpu.sync_copy(dis_v, dis_sh.at[pl.ds(s * RPS, RPS)])

    @pl.when(c == 0)
    def _sw():
        pltpu.sync_copy(sw_v, selfw_h.at[pl.ds(s * RPS, RPS)])
    plsc.subcore_barrier()

    pltpu.sync_copy(dis_sh, dis_f)

    @pl.loop(0, nb)
    def _nrm(r):
        for k in range(B // L):
            sl = pl.ds(k * L, L)
            s16 = src_v[r, sl]
            d16 = dst_v[c * nb + r, sl]
            e16 = ew_v[c * nb + r, sl]
            nrm_v[r, sl] = (plsc.load_gather(dis_f, [s16]) * e16 *
                            plsc.load_gather(dis_f, [d16]))
    pltpu.sync_copy(nrm_v, norm_h.at[pl.ds(rb + c * nb, nb)])


_norm_call = pl.kernel(
    _norm_body,
    out_type=(jax.ShapeDtypeStruct((ER, B), jnp.float32),
              jax.ShapeDtypeStruct((NP,), jnp.float32)),
    mesh=_MESH,
    compiler_params=pltpu.CompilerParams(needs_layout_passes=False, use_tc_tiling_on_sc=False),
    scratch_types=[
        pltpu.VMEM((ER // NS, B), jnp.int32),
        pltpu.VMEM((ER // NS, B), jnp.float32),
        pltpu.VMEM((ER // NS // NC, B), jnp.int32),
        pltpu.VMEM((NP,), jnp.float32),
        pltpu.VMEM((RPS,), jnp.float32),
        pltpu.VMEM((RPS,), jnp.float32),
        pltpu.VMEM((RPS,), jnp.float32),
        pltpu.VMEM((ER // NS // NC, B), jnp.float32),
        pltpu.VMEM_SHARED((NP,), jnp.float32),
        pltpu.VMEM_SHARED((NP,), jnp.float32),
        pltpu.SemaphoreType.DMA,
    ],
)


def _make_prop(nch, cw, edge_split):
    nbs = (ER // NC if edge_split else ER) // NS
    npart = NC if edge_split else nch
    K = 4

    def body(yf_h, src_h, dst_h, nrm_h, z_h,
             src_v, gidx_v, dst_v, nrm_v, rows_v, acc_sh, gsem, ssem):
        c = lax.axis_index("c")
        s = lax.axis_index("s")
        rb = (c * (ER // NC) + s * nbs) if edge_split else (s * nbs)
        pltpu.sync_copy(src_h.at[pl.ds(rb, nbs)], src_v)
        pltpu.sync_copy(dst_h.at[pl.ds(rb, nbs)], dst_v)
        pltpu.sync_copy(nrm_h.at[pl.ds(rb, nbs)], nrm_v)

        for i in range(1 if edge_split else nch // NC):
            if edge_split:
                j = jnp.int32(0)
                zslot = c
            else:
                j = i * NC + c
                zslot = j

            @pl.loop(0, nbs)
            def _gi(b):
                for k in range(B // L):
                    sl = pl.ds(k * L, L)
                    gidx_v[b, sl] = src_v[b, sl] + j * NP

            @pl.loop(0, B)
            def _zr(r):
                for k in range(cw // L):
                    rows_v[0, r, pl.ds(k * L, L)] = jnp.zeros((L,), jnp.float32)
            for t in range(RPS // B):
                pltpu.sync_copy(rows_v.at[0],
                                acc_sh.at[pl.ds(s * RPS + t * B, B), :])
            plsc.subcore_barrier()

            @pl.loop(0, nbs // K)
            def _grp(it):
                base = it * K
                gd = []
                for u in range(K):
                    @pl.when(it > 0)
                    def _drain():
                        pltpu.make_async_copy(
                            yf_h.at[pl.ds(0, B), :], rows_v.at[u],
                            ssem.at[u]).wait()
                    gd.append(pltpu.async_copy(
                        yf_h.at[gidx_v.at[base + u]], rows_v.at[u],
                        gsem.at[u]))
                for u in range(K):
                    gd[u].wait()

                    @pl.loop(0, B // L)
                    def _scale(g):
                        w16 = nrm_v[base + u, pl.ds(g * L, L)]
                        for t in range(L):
                            w = w16[t]
                            r = g * L + t
                            for k in range(cw // L):
                                sl = pl.ds(k * L, L)
                                rows_v[u, r, sl] = rows_v[u, r, sl] * w
                    pltpu.async_copy(rows_v.at[u], acc_sh.at[dst_v.at[base + u]],
                                     ssem.at[u], add=True)
            for u in range(K):
                pltpu.make_async_copy(yf_h.at[pl.ds(0, B), :], rows_v.at[u],
                                      ssem.at[u]).wait()
            plsc.subcore_barrier()
            pltpu.sync_copy(acc_sh.at[pl.ds(s * RPS, RPS), :],
                            z_h.at[zslot, pl.ds(s * RPS, RPS), :])

    return pl.kernel(
        body,
        out_type=jax.ShapeDtypeStruct((npart, NP, cw), jnp.float32),
        mesh=_MESH,
        compiler_params=pltpu.CompilerParams(needs_layout_passes=False, use_tc_tiling_on_sc=False),
        scratch_types=[
            pltpu.VMEM((nbs, B), jnp.int32),
            pltpu.VMEM((nbs, B), jnp.int32),
            pltpu.VMEM((nbs, B), jnp.int32),
            pltpu.VMEM((nbs, B), jnp.float32),
            pltpu.VMEM((K, B, cw), jnp.float32),
            pltpu.VMEM_SHARED((NP, cw), jnp.float32),
            pltpu.SemaphoreType.DMA((K,)),
            pltpu.SemaphoreType.DMA((K,)),
        ],
    )


_prop_512 = _make_prop(8, 64, False)
_prop_64 = _make_prop(1, 64, True)
_prop_256 = _make_prop(4, 64, False)


def _tc1(x, W):
    nch = W.shape[1] // 64

    def body(x_ref, w_ref, o_ref):
        y = jnp.dot(x_ref[...], w_ref[...], preferred_element_type=jnp.float32)
        for jj in range(nch):
            o_ref[jj] = y[:, jj * 64:(jj + 1) * 64]

    return pl.pallas_call(
        body,
        grid=(GR,),
        in_specs=[pl.BlockSpec((MROW, 256), lambda i: (i, 0)),
                  pl.BlockSpec((256, W.shape[1]), lambda i: (0, 0))],
        out_specs=pl.BlockSpec((nch, MROW, 64), lambda i: (0, i, 0)),
        out_shape=jax.ShapeDtypeStruct((nch, NP, 64), jnp.float32),
    )(x, W)


def _tc_layer(z, y, sw, bc, W, nch_in, nch_out, parts, want_h):
    P = z.shape[0]
    din = nch_in * 64
    dout = nch_out * 64

    def body(z_ref, y_ref, sw_ref, b_ref, w_ref, yo_ref, *h_ref):
        hs = []
        for kk in range(nch_in):
            if parts:
                zsum = z_ref[0] + z_ref[1]
            else:
                zsum = z_ref[kk]
            hs.append(zsum + sw_ref[...] * y_ref[kk] + b_ref[kk])
        h = hs[0] if nch_in == 1 else jnp.concatenate(hs, axis=-1)
        if want_h:
            h_ref[0][...] = h
        yn = jnp.dot(h, w_ref[...], preferred_element_type=jnp.float32)
        for jj in range(nch_out):
            yo_ref[jj] = yn[:, jj * 64:(jj + 1) * 64]

    in_specs = [
        pl.BlockSpec((P, MROW, 64), lambda i: (0, i, 0)),
        pl.BlockSpec((nch_in, MROW, 64), lambda i: (0, i, 0)),
        pl.BlockSpec((MROW, 1), lambda i: (i, 0)),
        pl.BlockSpec((nch_in, 1, 64), lambda i: (0, 0, 0)),
        pl.BlockSpec((din, dout), lambda i: (0, 0)),
    ]
    out_shape = [jax.ShapeDtypeStruct((nch_out, NP, 64), jnp.float32)]
    out_specs = [pl.BlockSpec((nch_out, MROW, 64), lambda i: (0, i, 0))]
    if want_h:
        out_shape.append(jax.ShapeDtypeStruct((NP, din), jnp.float32))
        out_specs.append(pl.BlockSpec((MROW, din), lambda i: (i, 0)))
    res = pl.pallas_call(
        body,
        grid=(GR,),
        in_specs=in_specs,
        out_specs=out_specs,
        out_shape=out_shape,
    )(z, y, sw, bc, W)
    return res if want_h else res[0]


def _tc_final(z, y, sw, bc, nch):
    def body(z_ref, y_ref, sw_ref, b_ref, h_ref):
        hs = [z_ref[kk] + sw_ref[...] * y_ref[kk] + b_ref[kk]
              for kk in range(nch)]
        h_ref[...] = jnp.concatenate(hs, axis=-1)

    return pl.pallas_call(
        body,
        grid=(GR,),
        in_specs=[pl.BlockSpec((nch, MROW, 64), lambda i: (0, i, 0)),
                  pl.BlockSpec((nch, MROW, 64), lambda i: (0, i, 0)),
                  pl.BlockSpec((MROW, 1), lambda i: (i, 0)),
                  pl.BlockSpec((nch, 1, 64), lambda i: (0, 0, 0))],
        out_specs=pl.BlockSpec((MROW, nch * 64), lambda i: (i, 0)),
        out_shape=jax.ShapeDtypeStruct((NP, nch * 64), jnp.float32),
    )(z, y, sw, bc)


def kernel(features, edge_index, edge_weight, W1, b1, W2, b2, W3, b3, W4, b4):
    src = edge_index[0].astype(jnp.int32)
    dst = edge_index[1].astype(jnp.int32)
    ew = edge_weight.astype(jnp.float32)
    pad = EP - E
    src2 = jnp.concatenate([src, jnp.zeros((pad,), jnp.int32)]).reshape(ER, B)
    dst2 = jnp.concatenate([dst, jnp.zeros((pad,), jnp.int32)]).reshape(ER, B)
    ew2 = jnp.concatenate([ew, jnp.zeros((pad,), jnp.float32)]).reshape(ER, B)
    xp = jnp.pad(features, ((0, NP - N), (0, 0)))

    norm2, selfw = _norm_call(src2, dst2, ew2)
    sw = selfw.reshape(NP, 1)

    y1 = _tc1(xp, W1)
    z1 = _prop_512(y1.reshape(8 * NP, 64), src2, dst2, norm2)
    y2 = _tc_layer(z1, y1, sw, b1.reshape(8, 1, 64), W2,
                   8, 1, parts=False, want_h=False)
    z2 = _prop_64(y2.reshape(NP, 64), src2, dst2, norm2)
    y3, h2 = _tc_layer(z2, y2, sw, b2.reshape(1, 1, 64), W3,
                       1, 8, parts=True, want_h=True)
    z3 = _prop_512(y3.reshape(8 * NP, 64), src2, dst2, norm2)
    y4 = _tc_layer(z3, y3, sw, b3.reshape(8, 1, 64), W4,
                   8, 4, parts=False, want_h=False)
    z4 = _prop_256(y4.reshape(4 * NP, 64), src2, dst2, norm2)
    h4 = _tc_final(z4, y4, sw, b4.reshape(4, 1, 64), 4)
    return h2[:N], h4[:N]
